# Initial kernel scaffold; baseline (speedup 1.0000x reference)
import jax
import jax.numpy as jnp
from jax.experimental import pallas as pl


def kernel(h, edge_index, e, snorm_n, snorm_e, W_embed, b_embed, pre_W, pre_b, post_W, post_b, W1, b1, W2, b2, W3, b3):
    raise NotImplementedError("write your pallas kernel here")


# speedup vs baseline: 4.5116x; 4.5116x over previous
import functools

import jax
import jax.numpy as jnp
from jax import lax
from jax.experimental import pallas as pl
from jax.experimental.pallas import tpu as pltpu
from jax.experimental.pallas import tpu_sc as plsc

_NC = 2
_NS = 16
_LANES = 16
_CHUNK = 80
_DRAIN = 16


def _dot(a, b):
    return jnp.dot(a, b, preferred_element_type=jnp.float32)


def _tc_embed_pre(h0, W_embed, b_embed, preWa, preWb, pre_b):
    N, D = h0.shape

    def body(h0_ref, we_ref, be_ref, wa_ref, wb_ref, pb_ref,
             h_out, a_out, b_out):
        h = _dot(h0_ref[...], we_ref[...]) + be_ref[...]
        h_out[...] = h
        a_out[...] = _dot(h, wa_ref[...]) + pb_ref[...]
        b_out[...] = _dot(h, wb_ref[...])

    out = [jax.ShapeDtypeStruct((N, D), jnp.float32)] * 3
    return pl.pallas_call(body, out_shape=out)(
        h0, W_embed, b_embed, preWa, preWb, pre_b)


def _tc_post_pre(h, psum, csum, snorm, postWh, postWm, post_b,
                 preWa, preWb, pre_b):
    N, D = h.shape

    def body(h_ref, p_ref, c_ref, sn_ref, wh_ref, wm_ref, pb_ref,
             wa_ref, wb_ref, prb_ref, h_out, a_out, b_out):
        summed = p_ref[:N] + p_ref[N:]
        cnt = c_ref[:N, 0:1] + c_ref[N:, 0:1]
        m = summed / jnp.maximum(cnt, 1.0)
        t = jnp.maximum(
            _dot(h_ref[...], wh_ref[...]) + _dot(m, wm_ref[...]) + pb_ref[...],
            0.0)
        h2 = h_ref[...] + t * sn_ref[...]
        h_out[...] = h2
        a_out[...] = _dot(h2, wa_ref[...]) + prb_ref[...]
        b_out[...] = _dot(h2, wb_ref[...])

    out = [jax.ShapeDtypeStruct((N, D), jnp.float32)] * 3
    return pl.pallas_call(body, out_shape=out)(
        h, psum, csum, snorm, postWh, postWm, post_b, preWa, preWb, pre_b)


def _tc_post_readout(h, psum, csum, snorm, postWh, postWm, post_b,
                     W1, b1, W2, b2, W3, b3):
    N, D = h.shape

    def body(h_ref, p_ref, c_ref, sn_ref, wh_ref, wm_ref, pb_ref,
             w1_ref, b1_ref, w2_ref, b2_ref, w3_ref, b3_ref, y_out):
        summed = p_ref[:N] + p_ref[N:]
        cnt = c_ref[:N, 0:1] + c_ref[N:, 0:1]
        m = summed / jnp.maximum(cnt, 1.0)
        t = jnp.maximum(
            _dot(h_ref[...], wh_ref[...]) + _dot(m, wm_ref[...]) + pb_ref[...],
            0.0)
        h2 = h_ref[...] + t * sn_ref[...]
        hg = jnp.mean(h2, axis=0, keepdims=True)
        y = jnp.maximum(_dot(hg, w1_ref[...]) + b1_ref[...], 0.0)
        y = jnp.maximum(_dot(y, w2_ref[...]) + b2_ref[...], 0.0)
        y_out[...] = _dot(y, w3_ref[...]) + b3_ref[...]

    out = jax.ShapeDtypeStruct((1, 1), jnp.float32)
    return pl.pallas_call(body, out_shape=out)(
        h, psum, csum, snorm, postWh, postWm, post_b,
        W1, b1, W2, b2, W3, b3)


def _sc_edge(A, B, src, dst):
    N, D = A.shape
    E = src.shape[0]
    n_chunks = E // _CHUNK
    per_core = n_chunks // _NC
    rows_base = (N // _NS) // 8 * 8
    rows_last = N - rows_base * (_NS - 1)

    mesh = plsc.VectorSubcoreMesh(core_axis_name="c", subcore_axis_name="s",
                                  num_cores=_NC, num_subcores=_NS)

    out_type = [jax.ShapeDtypeStruct((_NC * N, D), jnp.float32)]
    scratch = [
        pltpu.VMEM((_CHUNK,), jnp.int32),
        pltpu.VMEM((_CHUNK,), jnp.int32),
        pltpu.VMEM((_CHUNK, D), jnp.float32),
        pltpu.VMEM((_CHUNK, D), jnp.float32),
        pltpu.VMEM_SHARED((N, D), jnp.float32),
        pltpu.SemaphoreType.DMA,
        pltpu.SemaphoreType.DMA,
    ]

    def body(a_hbm, b_hbm, src_hbm, dst_hbm,
             out_hbm, srcv, dstv, ra, rb, acc, sA, sB):
        c = lax.axis_index("c")
        s = lax.axis_index("s")
        row0 = s * rows_base
        nrows = jnp.where(s == _NS - 1, rows_last, rows_base)
        n_drain = nrows // _DRAIN

        zvec = jnp.zeros((_LANES,), jnp.float32)

        def zrow(i, _):
            for k in range(D // _LANES):
                ra[i, pl.ds(k * _LANES, _LANES)] = zvec
            return 0
        lax.fori_loop(0, _DRAIN, zrow, 0)

        def zcopy(i, _):
            pltpu.sync_copy(ra.at[pl.ds(0, _DRAIN)],
                            acc.at[pl.ds(row0 + i * _DRAIN, _DRAIN)])
            return 0
        lax.fori_loop(0, n_drain, zcopy, 0)

        plsc.subcore_barrier()

        n_iter = (per_core - s + _NS - 1) // _NS

        def chunk(j, _):
            cid = c * per_core + s + j * _NS
            base = cid * _CHUNK
            pltpu.sync_copy(src_hbm.at[pl.ds(base, _CHUNK)], srcv)
            pltpu.sync_copy(dst_hbm.at[pl.ds(base, _CHUNK)], dstv)
            cpa = pltpu.async_copy(a_hbm.at[srcv], ra, sA)
            cpb = pltpu.async_copy(b_hbm.at[dstv], rb, sB)
            cpa.wait()
            cpb.wait()

            def vrow(r, _):
                for k in range(D // _LANES):
                    sl = pl.ds(k * _LANES, _LANES)
                    ra[r, sl] = jnp.maximum(ra[r, sl] + rb[r, sl], 0.0)
                return 0
            lax.fori_loop(0, _CHUNK, vrow, 0)

            pltpu.sync_copy(ra, acc.at[dstv], add=True)
            return 0
        lax.fori_loop(0, n_iter, chunk, 0)

        plsc.subcore_barrier()

        def dcopy(i, _):
            r = row0 + i * _DRAIN
            pltpu.sync_copy(acc.at[pl.ds(r, _DRAIN)], ra.at[pl.ds(0, _DRAIN)])
            pltpu.sync_copy(ra.at[pl.ds(0, _DRAIN)], out_hbm.at[pl.ds(c * N + r, _DRAIN)])
            return 0
        lax.fori_loop(0, n_drain, dcopy, 0)

    fn = pl.kernel(body, out_type=out_type, mesh=mesh, scratch_types=scratch)
    return fn(A, B, src, dst)[0]


def _sc_counts(dst, N, D):
    E = dst.shape[0]
    n_chunks = E // _CHUNK
    per_core = n_chunks // _NC
    rows_base = (N // _NS) // 8 * 8
    rows_last = N - rows_base * (_NS - 1)

    mesh = plsc.VectorSubcoreMesh(core_axis_name="c", subcore_axis_name="s",
                                  num_cores=_NC, num_subcores=_NS)

    out_type = [jax.ShapeDtypeStruct((_NC * N, D), jnp.float32)]
    scratch = [
        pltpu.VMEM((_CHUNK,), jnp.int32),
        pltpu.VMEM((_CHUNK, D), jnp.float32),
        pltpu.VMEM_SHARED((N, D), jnp.float32),
    ]

    def body(dst_hbm, cnt_hbm, dstv, ones, cacc):
        c = lax.axis_index("c")
        s = lax.axis_index("s")
        row0 = s * rows_base
        nrows = jnp.where(s == _NS - 1, rows_last, rows_base)
        n_drain = nrows // _DRAIN
        zvec = jnp.zeros((_LANES,), jnp.float32)
        one0 = jnp.maximum(
            1.0 - lax.iota(jnp.int32, _LANES).astype(jnp.float32), 0.0)

        def zrow(i, _):
            for k in range(D // _LANES):
                ones[i, pl.ds(k * _LANES, _LANES)] = zvec
            return 0
        lax.fori_loop(0, _CHUNK, zrow, 0)

        def zcopy(i, _):
            pltpu.sync_copy(ones.at[pl.ds(0, _DRAIN)],
                            cacc.at[pl.ds(row0 + i * _DRAIN, _DRAIN)])
            return 0
        lax.fori_loop(0, n_drain, zcopy, 0)

        def orow(i, _):
            ones[i, pl.ds(0, _LANES)] = one0
            return 0
        lax.fori_loop(0, _CHUNK, orow, 0)

        plsc.subcore_barrier()

        n_iter = (per_core - s + _NS - 1) // _NS

        def chunk(j, _):
            cid = c * per_core + s + j * _NS
            base = cid * _CHUNK
            pltpu.sync_copy(dst_hbm.at[pl.ds(base, _CHUNK)], dstv)
            pltpu.sync_copy(ones, cacc.at[dstv], add=True)
            return 0
        lax.fori_loop(0, n_iter, chunk, 0)

        plsc.subcore_barrier()

        def dccopy(i, _):
            r = row0 + i * _DRAIN
            pltpu.sync_copy(cacc.at[pl.ds(r, _DRAIN)], ones.at[pl.ds(0, _DRAIN)])
            pltpu.sync_copy(ones.at[pl.ds(0, _DRAIN)],
                            cnt_hbm.at[pl.ds(c * N + r, _DRAIN)])
            return 0
        lax.fori_loop(0, n_drain, dccopy, 0)

    fn = pl.kernel(body, out_type=out_type, mesh=mesh, scratch_types=scratch)
    return fn(dst)[0]


def kernel(h, edge_index, e, snorm_n, snorm_e, W_embed, b_embed,
           pre_W, pre_b, post_W, post_b, W1, b1, W2, b2, W3, b3):
    del e, snorm_e
    N, D = h.shape
    L = pre_W.shape[0]
    src = edge_index[0]
    dst = edge_index[1]

    b_embed2 = b_embed.reshape(1, D)
    h1, A, Bm = _tc_embed_pre(h, W_embed, b_embed2,
                              pre_W[0, :D], pre_W[0, D:],
                              pre_b[0].reshape(1, D))
    csum = _sc_counts(dst, N, D)
    psum = _sc_edge(A, Bm, src, dst)

    for i in range(1, L):
        h1, A, Bm = _tc_post_pre(
            h1, psum, csum, snorm_n,
            post_W[i - 1, :D], post_W[i - 1, D:], post_b[i - 1].reshape(1, D),
            pre_W[i, :D], pre_W[i, D:], pre_b[i].reshape(1, D))
        psum = _sc_edge(A, Bm, src, dst)

    y = _tc_post_readout(
        h1, psum, csum, snorm_n,
        post_W[L - 1, :D], post_W[L - 1, D:], post_b[L - 1].reshape(1, D),
        W1, b1.reshape(1, -1), W2, b2.reshape(1, -1), W3, b3.reshape(1, -1))
    return y


# speedup vs baseline: 6.6966x; 1.4843x over previous
import functools

import jax
import jax.numpy as jnp
from jax import lax
from jax.experimental import pallas as pl
from jax.experimental.pallas import tpu as pltpu
from jax.experimental.pallas import tpu_sc as plsc

_NC = 2
_NS = 16
_LANES = 16
_CHUNK = 80
_DRAIN = 16


def _dot(a, b):
    return jnp.dot(a, b, preferred_element_type=jnp.float32)


def _tc_embed_pre(h0, W_embed, b_embed, preWa, preWb, pre_b):
    N, D = h0.shape

    def body(h0_ref, we_ref, be_ref, wa_ref, wb_ref, pb_ref,
             h_out, a_out, b_out):
        h = _dot(h0_ref[...], we_ref[...]) + be_ref[...]
        h_out[...] = h
        a_out[...] = _dot(h, wa_ref[...]) + pb_ref[...]
        b_out[...] = _dot(h, wb_ref[...])

    out = [jax.ShapeDtypeStruct((N, D), jnp.float32)] * 3
    return pl.pallas_call(body, out_shape=out)(
        h0, W_embed, b_embed, preWa, preWb, pre_b)


def _tc_post_pre(h, psum, csum, snorm, postWh, postWm, post_b,
                 preWa, preWb, pre_b):
    N, D = h.shape

    def body(h_ref, p_ref, c_ref, sn_ref, wh_ref, wm_ref, pb_ref,
             wa_ref, wb_ref, prb_ref, h_out, a_out, b_out):
        summed = p_ref[:N] + p_ref[N:]
        cnt = c_ref[:N, 0:1] + c_ref[N:, 0:1]
        m = summed / jnp.maximum(cnt, 1.0)
        t = jnp.maximum(
            _dot(h_ref[...], wh_ref[...]) + _dot(m, wm_ref[...]) + pb_ref[...],
            0.0)
        h2 = h_ref[...] + t * sn_ref[...]
        h_out[...] = h2
        a_out[...] = _dot(h2, wa_ref[...]) + prb_ref[...]
        b_out[...] = _dot(h2, wb_ref[...])

    out = [jax.ShapeDtypeStruct((N, D), jnp.float32)] * 3
    return pl.pallas_call(body, out_shape=out)(
        h, psum, csum, snorm, postWh, postWm, post_b, preWa, preWb, pre_b)


def _tc_post_readout(h, psum, csum, snorm, postWh, postWm, post_b,
                     W1, b1, W2, b2, W3, b3):
    N, D = h.shape

    def body(h_ref, p_ref, c_ref, sn_ref, wh_ref, wm_ref, pb_ref,
             w1_ref, b1_ref, w2_ref, b2_ref, w3_ref, b3_ref, y_out):
        summed = p_ref[:N] + p_ref[N:]
        cnt = c_ref[:N, 0:1] + c_ref[N:, 0:1]
        m = summed / jnp.maximum(cnt, 1.0)
        t = jnp.maximum(
            _dot(h_ref[...], wh_ref[...]) + _dot(m, wm_ref[...]) + pb_ref[...],
            0.0)
        h2 = h_ref[...] + t * sn_ref[...]
        hg = jnp.mean(h2, axis=0, keepdims=True)
        y = jnp.maximum(_dot(hg, w1_ref[...]) + b1_ref[...], 0.0)
        y = jnp.maximum(_dot(y, w2_ref[...]) + b2_ref[...], 0.0)
        y_out[...] = _dot(y, w3_ref[...]) + b3_ref[...]

    out = jax.ShapeDtypeStruct((1, 1), jnp.float32)
    return pl.pallas_call(body, out_shape=out)(
        h, psum, csum, snorm, postWh, postWm, post_b,
        W1, b1, W2, b2, W3, b3)


_NBUF = 2


def _sc_edge(A, B, src, dst):
    N, D = A.shape
    E = src.shape[0]
    n_chunks = E // _CHUNK
    per_core = n_chunks // _NC
    per_sub = per_core // _NS
    rows_base = (N // _NS) // 8 * 8
    rows_last = N - rows_base * (_NS - 1)

    mesh = plsc.VectorSubcoreMesh(core_axis_name="c", subcore_axis_name="s",
                                  num_cores=_NC, num_subcores=_NS)

    out_type = [jax.ShapeDtypeStruct((_NC * N, D), jnp.float32)]
    scratch = (
        [pltpu.VMEM((_CHUNK,), jnp.int32)] * (2 * _NBUF) +
        [pltpu.VMEM((_CHUNK, D), jnp.float32)] * (2 * _NBUF) +
        [pltpu.VMEM_SHARED((N, D), jnp.float32)] +
        [pltpu.SemaphoreType.DMA] * (2 * _NBUF)
    )

    def body(a_hbm, b_hbm, src_hbm, dst_hbm, out_hbm, *rest):
        srcv = rest[0:_NBUF]
        dstv = rest[_NBUF:2 * _NBUF]
        ra = rest[2 * _NBUF:3 * _NBUF]
        rb = rest[3 * _NBUF:4 * _NBUF]
        acc = rest[4 * _NBUF]
        sA = rest[4 * _NBUF + 1:5 * _NBUF + 1]
        sB = rest[5 * _NBUF + 1:6 * _NBUF + 1]
        c = lax.axis_index("c")
        s = lax.axis_index("s")
        row0 = s * rows_base
        nrows = jnp.where(s == _NS - 1, rows_last, rows_base)
        n_drain = nrows // _DRAIN

        zvec = jnp.zeros((_LANES,), jnp.float32)

        def zrow(i, _):
            for k in range(D // _LANES):
                ra[0][i, pl.ds(k * _LANES, _LANES)] = zvec
            return 0
        lax.fori_loop(0, _DRAIN, zrow, 0)

        def zcopy(i, _):
            pltpu.sync_copy(ra[0].at[pl.ds(0, _DRAIN)],
                            acc.at[pl.ds(row0 + i * _DRAIN, _DRAIN)])
            return 0
        lax.fori_loop(0, n_drain, zcopy, 0)

        plsc.subcore_barrier()

        def base_of(j):
            return (c * per_core + s + j * _NS) * _CHUNK

        def start(b, j):
            base = base_of(j)
            pltpu.sync_copy(src_hbm.at[pl.ds(base, _CHUNK)], srcv[b])
            pltpu.sync_copy(dst_hbm.at[pl.ds(base, _CHUNK)], dstv[b])
            pltpu.async_copy(a_hbm.at[srcv[b]], ra[b], sA[b])
            pltpu.async_copy(b_hbm.at[dstv[b]], rb[b], sB[b])

        def finish(b):
            pltpu.make_async_copy(a_hbm.at[srcv[b]], ra[b], sA[b]).wait()
            pltpu.make_async_copy(b_hbm.at[dstv[b]], rb[b], sB[b]).wait()

            def vrow(r, _):
                for k in range(D // _LANES):
                    sl = pl.ds(k * _LANES, _LANES)
                    ra[b][r, sl] = jnp.maximum(ra[b][r, sl] + rb[b][r, sl], 0.0)
                return 0
            lax.fori_loop(0, _CHUNK, vrow, 0)

            pltpu.sync_copy(ra[b], acc.at[dstv[b]], add=True)

        start(0, 0)
        start(1, 1)

        def main(g, _):
            for b in range(_NBUF):
                finish(b)
                start(b, g * _NBUF + b + _NBUF)
            return 0
        lax.fori_loop(0, (per_sub - 3) // _NBUF, main, 0)

        finish(0)
        start(0, per_sub - 1)
        finish(1)
        finish(0)

        plsc.subcore_barrier()

        def dcopy(i, _):
            r = row0 + i * _DRAIN
            pltpu.sync_copy(acc.at[pl.ds(r, _DRAIN)], ra[0].at[pl.ds(0, _DRAIN)])
            pltpu.sync_copy(ra[0].at[pl.ds(0, _DRAIN)],
                            out_hbm.at[pl.ds(c * N + r, _DRAIN)])
            return 0
        lax.fori_loop(0, n_drain, dcopy, 0)

    fn = pl.kernel(body, out_type=out_type, mesh=mesh, scratch_types=scratch)
    return fn(A, B, src, dst)[0]


def _sc_counts(dst, N, D):
    E = dst.shape[0]
    n_chunks = E // _CHUNK
    per_core = n_chunks // _NC
    rows_base = (N // _NS) // 8 * 8
    rows_last = N - rows_base * (_NS - 1)

    mesh = plsc.VectorSubcoreMesh(core_axis_name="c", subcore_axis_name="s",
                                  num_cores=_NC, num_subcores=_NS)

    out_type = [jax.ShapeDtypeStruct((_NC * N, D), jnp.float32)]
    scratch = [
        pltpu.VMEM((_CHUNK,), jnp.int32),
        pltpu.VMEM((_CHUNK, D), jnp.float32),
        pltpu.VMEM_SHARED((N, D), jnp.float32),
    ]

    def body(dst_hbm, cnt_hbm, dstv, ones, cacc):
        c = lax.axis_index("c")
        s = lax.axis_index("s")
        row0 = s * rows_base
        nrows = jnp.where(s == _NS - 1, rows_last, rows_base)
        n_drain = nrows // _DRAIN
        zvec = jnp.zeros((_LANES,), jnp.float32)
        one0 = jnp.maximum(
            1.0 - lax.iota(jnp.int32, _LANES).astype(jnp.float32), 0.0)

        def zrow(i, _):
            for k in range(D // _LANES):
                ones[i, pl.ds(k * _LANES, _LANES)] = zvec
            return 0
        lax.fori_loop(0, _CHUNK, zrow, 0)

        def zcopy(i, _):
            pltpu.sync_copy(ones.at[pl.ds(0, _DRAIN)],
                            cacc.at[pl.ds(row0 + i * _DRAIN, _DRAIN)])
            return 0
        lax.fori_loop(0, n_drain, zcopy, 0)

        def orow(i, _):
            ones[i, pl.ds(0, _LANES)] = one0
            return 0
        lax.fori_loop(0, _CHUNK, orow, 0)

        plsc.subcore_barrier()

        n_iter = (per_core - s + _NS - 1) // _NS

        def chunk(j, _):
            cid = c * per_core + s + j * _NS
            base = cid * _CHUNK
            pltpu.sync_copy(dst_hbm.at[pl.ds(base, _CHUNK)], dstv)
            pltpu.sync_copy(ones, cacc.at[dstv], add=True)
            return 0
        lax.fori_loop(0, n_iter, chunk, 0)

        plsc.subcore_barrier()

        def dccopy(i, _):
            r = row0 + i * _DRAIN
            pltpu.sync_copy(cacc.at[pl.ds(r, _DRAIN)], ones.at[pl.ds(0, _DRAIN)])
            pltpu.sync_copy(ones.at[pl.ds(0, _DRAIN)],
                            cnt_hbm.at[pl.ds(c * N + r, _DRAIN)])
            return 0
        lax.fori_loop(0, n_drain, dccopy, 0)

    fn = pl.kernel(body, out_type=out_type, mesh=mesh, scratch_types=scratch)
    return fn(dst)[0]


def kernel(h, edge_index, e, snorm_n, snorm_e, W_embed, b_embed,
           pre_W, pre_b, post_W, post_b, W1, b1, W2, b2, W3, b3):
    del e, snorm_e
    N, D = h.shape
    L = pre_W.shape[0]
    src = edge_index[0]
    dst = edge_index[1]

    b_embed2 = b_embed.reshape(1, D)
    h1, A, Bm = _tc_embed_pre(h, W_embed, b_embed2,
                              pre_W[0, :D], pre_W[0, D:],
                              pre_b[0].reshape(1, D))
    csum = _sc_counts(dst, N, D)
    psum = _sc_edge(A, Bm, src, dst)

    for i in range(1, L):
        h1, A, Bm = _tc_post_pre(
            h1, psum, csum, snorm_n,
            post_W[i - 1, :D], post_W[i - 1, D:], post_b[i - 1].reshape(1, D),
            pre_W[i, :D], pre_W[i, D:], pre_b[i].reshape(1, D))
        psum = _sc_edge(A, Bm, src, dst)

    y = _tc_post_readout(
        h1, psum, csum, snorm_n,
        post_W[L - 1, :D], post_W[L - 1, D:], post_b[L - 1].reshape(1, D),
        W1, b1.reshape(1, -1), W2, b2.reshape(1, -1), W3, b3.reshape(1, -1))
    return y
